# baseline (speedup 1.0000x reference)
import jax
import jax.numpy as jnp
from jax.experimental import pallas as pl
from jax.experimental.pallas import tpu as pltpu

_TN = 512


def _sym_loss_kernel(a_ref, bt_ref, u_ref, v_ref, out_ref):
    b = pl.program_id(0)
    t = pl.program_id(1)

    @pl.when(jnp.logical_and(b == 0, t == 0))
    def _init():
        out_ref[0, 0] = 0.0

    ab = jnp.dot(a_ref[0], bt_ref[0], preferred_element_type=jnp.float32)
    nn = jnp.dot(u_ref[0], v_ref[0], preferred_element_type=jnp.float32,
                 precision=jax.lax.Precision.HIGHEST)
    d2 = nn - 2.0 * ab
    m = jnp.min(d2, axis=1, keepdims=True)
    out_ref[0, 0] += jnp.sum(m)


def kernel(xyz):
    B, N, _ = xyz.shape
    x = xyz[..., 0]
    y = xyz[..., 1]
    z = xyz[..., 2]
    n = x * x + y * y + z * z
    ones = jnp.ones_like(n)
    zeros = jnp.zeros_like(n)
    a = jnp.stack([-x, y, z, zeros, zeros, zeros, zeros, zeros], axis=-1)
    bt = jnp.stack([x, y, z, zeros, zeros, zeros, zeros, zeros], axis=1)
    u = jnp.stack([n, ones, zeros, zeros, zeros, zeros, zeros, zeros], axis=-1)
    v = jnp.stack([ones, n, zeros, zeros, zeros, zeros, zeros, zeros], axis=1)

    total = pl.pallas_call(
        _sym_loss_kernel,
        grid=(B, N // _TN),
        in_specs=[
            pl.BlockSpec((1, _TN, 8), lambda b, t: (b, t, 0)),
            pl.BlockSpec((1, 8, N), lambda b, t: (b, 0, 0)),
            pl.BlockSpec((1, _TN, 8), lambda b, t: (b, t, 0)),
            pl.BlockSpec((1, 8, N), lambda b, t: (b, 0, 0)),
        ],
        out_specs=pl.BlockSpec(memory_space=pltpu.SMEM),
        out_shape=jax.ShapeDtypeStruct((1, 1), jnp.float32),
    )(a, bt, u, v)
    return total[0, 0] * (2.0 / (B * N))


# speedup vs baseline: 4.6250x; 4.6250x over previous
import jax
import jax.numpy as jnp
from jax.experimental import pallas as pl
from jax.experimental.pallas import tpu as pltpu

_TN = 512


def _sym_loss_kernel(a_ref, bt_ref, out_ref):
    b = pl.program_id(0)
    t = pl.program_id(1)

    @pl.when(jnp.logical_and(b == 0, t == 0))
    def _init():
        out_ref[0, 0] = 0.0

    a = a_ref[0]
    bt = bt_ref[0]
    ab = jnp.dot(a, bt, preferred_element_type=jnp.float32)
    nrow = bt_ref[0, 3:4, :]
    e = nrow - 2.0 * ab
    m = jnp.min(e, axis=1, keepdims=True)
    acc = jnp.sum(m)

    @pl.when(t == 0)
    def _add_norms():
        out_ref[0, 0] += jnp.sum(nrow)

    out_ref[0, 0] += acc


def kernel(xyz):
    B, N, _ = xyz.shape
    x = xyz[..., 0]
    y = xyz[..., 1]
    z = xyz[..., 2]
    n = x * x + y * y + z * z
    zeros = jnp.zeros_like(n)
    a = jnp.stack([-x, y, z, zeros, zeros, zeros, zeros, zeros], axis=-1)
    bt = jnp.stack([x, y, z, n, zeros, zeros, zeros, zeros], axis=1)

    total = pl.pallas_call(
        _sym_loss_kernel,
        grid=(B, N // _TN),
        in_specs=[
            pl.BlockSpec((1, _TN, 8), lambda b, t: (b, t, 0)),
            pl.BlockSpec((1, 8, N), lambda b, t: (b, 0, 0)),
        ],
        out_specs=pl.BlockSpec(memory_space=pltpu.SMEM),
        out_shape=jax.ShapeDtypeStruct((1, 1), jnp.float32),
    )(a, bt)
    return total[0, 0] * (2.0 / (B * N))


# speedup vs baseline: 4.6764x; 1.0111x over previous
import jax
import jax.numpy as jnp
from jax.experimental import pallas as pl
from jax.experimental.pallas import tpu as pltpu

_TN = 512


def _sym_loss_kernel(a_ref, bt_ref, out_ref):
    b = pl.program_id(0)
    t = pl.program_id(1)

    @pl.when(jnp.logical_and(b == 0, t == 0))
    def _init():
        out_ref[0, 0] = 0.0

    a = a_ref[0]
    bt = bt_ref[0]
    ab2 = jnp.dot(a, bt, preferred_element_type=jnp.float32)
    nrow = bt_ref[0, 3:4, :]
    e = nrow - ab2
    m = jnp.min(e, axis=1, keepdims=True)
    acc = jnp.sum(m)

    @pl.when(t == 0)
    def _add_norms():
        out_ref[0, 0] += jnp.sum(nrow)

    out_ref[0, 0] += acc


def kernel(xyz):
    B, N, _ = xyz.shape
    x = xyz[..., 0]
    y = xyz[..., 1]
    z = xyz[..., 2]
    n = x * x + y * y + z * z
    zeros = jnp.zeros_like(n)
    a = jnp.stack([-2 * x, 2 * y, 2 * z, zeros, zeros, zeros, zeros, zeros],
                  axis=-1)
    bt = jnp.stack([x, y, z, n, zeros, zeros, zeros, zeros], axis=1)

    total = pl.pallas_call(
        _sym_loss_kernel,
        grid=(B, N // _TN),
        in_specs=[
            pl.BlockSpec((1, _TN, 8), lambda b, t: (b, t, 0)),
            pl.BlockSpec((1, 8, N), lambda b, t: (b, 0, 0)),
        ],
        out_specs=pl.BlockSpec(memory_space=pltpu.SMEM),
        out_shape=jax.ShapeDtypeStruct((1, 1), jnp.float32),
    )(a, bt)
    return total[0, 0] * (2.0 / (B * N))


# speedup vs baseline: 5.2107x; 1.1143x over previous
import jax
import jax.numpy as jnp
from jax.experimental import pallas as pl
from jax.experimental.pallas import tpu as pltpu

_TN = 1024


def _sym_loss_kernel(a_ref, bt_ref, out_ref):
    b = pl.program_id(0)
    t = pl.program_id(1)

    @pl.when(jnp.logical_and(b == 0, t == 0))
    def _init():
        out_ref[0, 0] = 0.0

    a = a_ref[0]
    bt = bt_ref[0]
    ab2 = jnp.dot(a, bt, preferred_element_type=jnp.float32)
    nrow = bt_ref[0, 3:4, :]
    e = nrow - ab2
    m = jnp.min(e, axis=1, keepdims=True)
    acc = jnp.sum(m)

    @pl.when(t == 0)
    def _add_norms():
        out_ref[0, 0] += jnp.sum(nrow)

    out_ref[0, 0] += acc


def kernel(xyz):
    B, N, _ = xyz.shape
    x = xyz[..., 0]
    y = xyz[..., 1]
    z = xyz[..., 2]
    n = x * x + y * y + z * z
    zeros = jnp.zeros_like(n)
    a = jnp.stack([-2 * x, 2 * y, 2 * z, zeros, zeros, zeros, zeros, zeros],
                  axis=-1)
    bt = jnp.stack([x, y, z, n, zeros, zeros, zeros, zeros], axis=1)

    total = pl.pallas_call(
        _sym_loss_kernel,
        grid=(B, N // _TN),
        in_specs=[
            pl.BlockSpec((1, _TN, 8), lambda b, t: (b, t, 0)),
            pl.BlockSpec((1, 8, N), lambda b, t: (b, 0, 0)),
        ],
        out_specs=pl.BlockSpec(memory_space=pltpu.SMEM),
        out_shape=jax.ShapeDtypeStruct((1, 1), jnp.float32),
    )(a, bt)
    return total[0, 0] * (2.0 / (B * N))


# speedup vs baseline: 5.5108x; 1.0576x over previous
import jax
import jax.numpy as jnp
from jax.experimental import pallas as pl
from jax.experimental.pallas import tpu as pltpu

_TN = 2048


def _sym_loss_kernel(a_ref, bt_ref, out_ref):
    b = pl.program_id(0)
    t = pl.program_id(1)

    @pl.when(jnp.logical_and(b == 0, t == 0))
    def _init():
        out_ref[0, 0] = 0.0

    a = a_ref[0]
    bt = bt_ref[0]
    ab2 = jnp.dot(a, bt, preferred_element_type=jnp.float32)
    nrow = bt_ref[0, 3:4, :]
    e = nrow - ab2
    m = jnp.min(e, axis=1, keepdims=True)
    acc = jnp.sum(m)

    @pl.when(t == 0)
    def _add_norms():
        out_ref[0, 0] += jnp.sum(nrow)

    out_ref[0, 0] += acc


def kernel(xyz):
    B, N, _ = xyz.shape
    x = xyz[..., 0]
    y = xyz[..., 1]
    z = xyz[..., 2]
    n = x * x + y * y + z * z
    zeros = jnp.zeros_like(n)
    a = jnp.stack([-2 * x, 2 * y, 2 * z, zeros, zeros, zeros, zeros, zeros],
                  axis=-1)
    bt = jnp.stack([x, y, z, n, zeros, zeros, zeros, zeros], axis=1)

    total = pl.pallas_call(
        _sym_loss_kernel,
        grid=(B, N // _TN),
        in_specs=[
            pl.BlockSpec((1, _TN, 8), lambda b, t: (b, t, 0)),
            pl.BlockSpec((1, 8, N), lambda b, t: (b, 0, 0)),
        ],
        out_specs=pl.BlockSpec(memory_space=pltpu.SMEM),
        out_shape=jax.ShapeDtypeStruct((1, 1), jnp.float32),
    )(a, bt)
    return total[0, 0] * (2.0 / (B * N))


# speedup vs baseline: 5.6895x; 1.0324x over previous
import jax
import jax.numpy as jnp
from jax.experimental import pallas as pl
from jax.experimental.pallas import tpu as pltpu

_TN = 4096


def _sym_loss_kernel(a_ref, bt_ref, out_ref):
    b = pl.program_id(0)
    t = pl.program_id(1)

    @pl.when(jnp.logical_and(b == 0, t == 0))
    def _init():
        out_ref[0, 0] = 0.0

    a = a_ref[0]
    bt = bt_ref[0]
    ab2 = jnp.dot(a, bt, preferred_element_type=jnp.float32)
    nrow = bt_ref[0, 3:4, :]
    e = nrow - ab2
    m = jnp.min(e, axis=1, keepdims=True)
    acc = jnp.sum(m)

    @pl.when(t == 0)
    def _add_norms():
        out_ref[0, 0] += jnp.sum(nrow)

    out_ref[0, 0] += acc


def kernel(xyz):
    B, N, _ = xyz.shape
    x = xyz[..., 0]
    y = xyz[..., 1]
    z = xyz[..., 2]
    n = x * x + y * y + z * z
    zeros = jnp.zeros_like(n)
    a = jnp.stack([-2 * x, 2 * y, 2 * z, zeros, zeros, zeros, zeros, zeros],
                  axis=-1)
    bt = jnp.stack([x, y, z, n, zeros, zeros, zeros, zeros], axis=1)

    total = pl.pallas_call(
        _sym_loss_kernel,
        grid=(B, N // _TN),
        in_specs=[
            pl.BlockSpec((1, _TN, 8), lambda b, t: (b, t, 0)),
            pl.BlockSpec((1, 8, N), lambda b, t: (b, 0, 0)),
        ],
        out_specs=pl.BlockSpec(memory_space=pltpu.SMEM),
        out_shape=jax.ShapeDtypeStruct((1, 1), jnp.float32),
    )(a, bt)
    return total[0, 0] * (2.0 / (B * N))


# speedup vs baseline: 5.7678x; 1.0138x over previous
import jax
import jax.numpy as jnp
from jax.experimental import pallas as pl
from jax.experimental.pallas import tpu as pltpu

_TN = 4096


def _sym_loss_kernel(a_ref, bt_ref, out_ref):
    b = pl.program_id(0)
    t = pl.program_id(1)

    @pl.when(jnp.logical_and(b == 0, t == 0))
    def _init():
        out_ref[0, 0] = 0.0

    a = a_ref[0]
    bt = bt_ref[0]
    ab2 = jnp.dot(a, bt, preferred_element_type=jnp.float32)
    nrow = bt_ref[0, 3:4, :]
    e = nrow - ab2
    m = jnp.min(e, axis=1, keepdims=True)
    acc = jnp.sum(m)

    @pl.when(t == 0)
    def _add_norms():
        out_ref[0, 0] += jnp.sum(nrow)

    out_ref[0, 0] += acc


def kernel(xyz):
    B, N, _ = xyz.shape
    x = xyz[..., 0]
    y = xyz[..., 1]
    z = xyz[..., 2]
    n = x * x + y * y + z * z
    zeros = jnp.zeros_like(n)
    a = jnp.stack([-2 * x, 2 * y, 2 * z, zeros], axis=-1)
    bt = jnp.stack([x, y, z, n], axis=1)

    total = pl.pallas_call(
        _sym_loss_kernel,
        grid=(B, N // _TN),
        in_specs=[
            pl.BlockSpec((1, _TN, 4), lambda b, t: (b, t, 0)),
            pl.BlockSpec((1, 4, N), lambda b, t: (b, 0, 0)),
        ],
        out_specs=pl.BlockSpec(memory_space=pltpu.SMEM),
        out_shape=jax.ShapeDtypeStruct((1, 1), jnp.float32),
    )(a, bt)
    return total[0, 0] * (2.0 / (B * N))
